# Initial kernel scaffold; baseline (speedup 1.0000x reference)
import jax
import jax.numpy as jnp
from jax.experimental import pallas as pl


def kernel(x, edge_index, edge_attr, W1, b1, W2, b2):
    raise NotImplementedError("write your pallas kernel here")


# speedup vs baseline: 4.1621x; 4.1621x over previous
import functools

import jax
import jax.numpy as jnp
from jax import lax
from jax.experimental import pallas as pl
from jax.experimental.pallas import tpu as pltpu
from jax.experimental.pallas import tpu_sc as plsc

N_NODES = 10000
N_EDGES = 320000
D_FEAT = 128
D_EDGE = 16
HIDDEN = 128
OUT = 128

NC = 2
NS = 16
DH = D_FEAT // NC
E_PER_W = N_EDGES // NS
CHUNK = 80
N_CHUNKS = E_PER_W // CHUNK
EDGE_HALF = N_CHUNKS // NC
NZ_S = 10
ROWS_PER_S = N_NODES // NZ_S
ZROWS = 200


def _sc_agg_body(rxlo_hbm, rxhi_hbm, src_hbm, dst_hbm, ea_hbm,
                 outn_hbm, oute_hbm,
                 src_v, dst_v, rows_v, ebuf, zbuf, zbuf16,
                 accn_sh, acce_sh, sem):
    cid = lax.axis_index("c")
    sid = lax.axis_index("s")

    zero16 = jnp.zeros((16,), jnp.float32)

    def _zrow(i, carry):
        for k in range(DH // 16):
            zbuf[i, pl.ds(k * 16, 16)] = zero16
        zbuf16[i, :] = zero16
        return carry

    @pl.when(sid < NZ_S)
    def _zero_fill():
        lax.fori_loop(0, ZROWS, _zrow, 0)
        for j in range(ROWS_PER_S // ZROWS):
            r0 = sid * ROWS_PER_S + j * ZROWS
            pltpu.sync_copy(zbuf, accn_sh.at[pl.ds(r0, ZROWS)])
            pltpu.sync_copy(zbuf16, acce_sh.at[pl.ds(r0, ZROWS)])

    plsc.subcore_barrier()

    pltpu.sync_copy(src_hbm.at[sid], src_v)
    pltpu.sync_copy(dst_hbm.at[sid], dst_v)

    def _chunk(j, carry):
        @pl.when(cid == 0)
        def _g_lo():
            pltpu.async_copy(rxlo_hbm.at[src_v.at[j]], rows_v, sem).wait()

        @pl.when(cid == 1)
        def _g_hi():
            pltpu.async_copy(rxhi_hbm.at[src_v.at[j]], rows_v, sem).wait()
        pltpu.sync_copy(rows_v, accn_sh.at[dst_v.at[j]], add=True)

        @pl.when((j // EDGE_HALF) == cid)
        def _edge_part():
            pltpu.sync_copy(ea_hbm.at[sid, j], ebuf)
            for i in range(CHUNK):
                ebuf[i, :] = jnp.maximum(ebuf[i, :], 0.0)
            pltpu.sync_copy(ebuf, acce_sh.at[dst_v.at[j]], add=True)

        return carry

    lax.fori_loop(0, N_CHUNKS, _chunk, 0)
    plsc.subcore_barrier()

    @pl.when(sid < NZ_S)
    def _copy_out():
        r0 = sid * ROWS_PER_S
        pltpu.sync_copy(accn_sh.at[pl.ds(r0, ROWS_PER_S)],
                        outn_hbm.at[cid, pl.ds(r0, ROWS_PER_S)])
        pltpu.sync_copy(acce_sh.at[pl.ds(r0, ROWS_PER_S)],
                        oute_hbm.at[cid, pl.ds(r0, ROWS_PER_S)])


_sc_agg = functools.partial(
    pl.kernel,
    out_type=(jax.ShapeDtypeStruct((NC, N_NODES, DH), jnp.float32),
              jax.ShapeDtypeStruct((NC, N_NODES, D_EDGE), jnp.float32)),
    mesh=plsc.VectorSubcoreMesh(core_axis_name="c", subcore_axis_name="s"),
    compiler_params=pltpu.CompilerParams(use_tc_tiling_on_sc=False),
    scratch_types=[
        pltpu.VMEM((N_CHUNKS, CHUNK), jnp.int32),
        pltpu.VMEM((N_CHUNKS, CHUNK), jnp.int32),
        pltpu.VMEM((CHUNK, DH), jnp.float32),
        pltpu.VMEM((CHUNK, D_EDGE), jnp.float32),
        pltpu.VMEM((ZROWS, DH), jnp.float32),
        pltpu.VMEM((ZROWS, D_EDGE), jnp.float32),
        pltpu.VMEM_SHARED((N_NODES, DH), jnp.float32),
        pltpu.VMEM_SHARED((N_NODES, D_EDGE), jnp.float32),
        pltpu.SemaphoreType.DMA,
    ],
)(_sc_agg_body)


def _relu_split_body(x_ref, olo_ref, ohi_ref):
    rx = jnp.maximum(x_ref[...], 0.0)
    olo_ref[...] = rx[:, :DH]
    ohi_ref[...] = rx[:, DH:]


def _mlp_body(an0, an1, ae0, ae1, xlo, xhi, w1lo, w1hi, w1e, b1, w2, b2,
              o_ref):
    hlo = an0[0] + xlo[...]
    hhi = an1[0] + xhi[...]
    he = ae0[0] + ae1[0]
    h1 = jnp.dot(hlo, w1lo[...], preferred_element_type=jnp.float32)
    h1 += jnp.dot(hhi, w1hi[...], preferred_element_type=jnp.float32)
    h1 += jnp.dot(he, w1e[...], preferred_element_type=jnp.float32)
    h1 = jnp.maximum(h1 + b1[...], 0.0)
    o_ref[...] = jnp.dot(h1, w2[...], preferred_element_type=jnp.float32) + b2[...]


def kernel(x, edge_index, edge_attr, W1, b1, W2, b2):
    src = edge_index[0].astype(jnp.int32).reshape(NS, N_CHUNKS, CHUNK)
    dst = edge_index[1].astype(jnp.int32).reshape(NS, N_CHUNKS, CHUNK)
    ea = edge_attr.reshape(NS, N_CHUNKS, CHUNK, D_EDGE)

    rx_lo, rx_hi = pl.pallas_call(
        _relu_split_body,
        out_shape=(jax.ShapeDtypeStruct((N_NODES, DH), jnp.float32),
                   jax.ShapeDtypeStruct((N_NODES, DH), jnp.float32)),
    )(x)

    accn, acce = _sc_agg(rx_lo, rx_hi, src, dst, ea)

    rows_blk = 1000
    grid = (N_NODES // rows_blk,)
    h_spec = pl.BlockSpec((1, rows_blk, DH), lambda i: (0, i, 0))
    h_spec1 = pl.BlockSpec((1, rows_blk, DH), lambda i: (1, i, 0))
    e_spec = pl.BlockSpec((1, rows_blk, D_EDGE), lambda i: (0, i, 0))
    e_spec1 = pl.BlockSpec((1, rows_blk, D_EDGE), lambda i: (1, i, 0))
    x_spec = pl.BlockSpec((rows_blk, DH), lambda i: (i, 0))
    out = pl.pallas_call(
        _mlp_body,
        out_shape=jax.ShapeDtypeStruct((N_NODES, OUT), jnp.float32),
        grid=grid,
        in_specs=[
            h_spec, h_spec1, e_spec, e_spec1, x_spec, x_spec,
            pl.BlockSpec((DH, HIDDEN), lambda i: (0, 0)),
            pl.BlockSpec((DH, HIDDEN), lambda i: (0, 0)),
            pl.BlockSpec((D_EDGE, HIDDEN), lambda i: (0, 0)),
            pl.BlockSpec((1, HIDDEN), lambda i: (0, 0)),
            pl.BlockSpec((HIDDEN, OUT), lambda i: (0, 0)),
            pl.BlockSpec((1, OUT), lambda i: (0, 0)),
        ],
        out_specs=pl.BlockSpec((rows_blk, OUT), lambda i: (i, 0)),
    )(accn, accn, acce, acce, x[:, :DH], x[:, DH:],
      W1[:DH], W1[DH:D_FEAT], W1[D_FEAT:], b1.reshape(1, HIDDEN),
      W2, b2.reshape(1, OUT))
    return out


# speedup vs baseline: 6.6556x; 1.5991x over previous
import functools

import jax
import jax.numpy as jnp
from jax import lax
from jax.experimental import pallas as pl
from jax.experimental.pallas import tpu as pltpu
from jax.experimental.pallas import tpu_sc as plsc

N_NODES = 10000
N_EDGES = 320000
D_FEAT = 128
D_EDGE = 16
HIDDEN = 128
OUT = 128

NC = 2
NS = 16
DH = D_FEAT // NC
E_PER_W = N_EDGES // NS
CHUNK = 80
N_CHUNKS = E_PER_W // CHUNK
EDGE_HALF = N_CHUNKS // NC
NZ_S = 10
ROWS_PER_S = N_NODES // NZ_S
ZROWS = 200


def _sc_agg_body(rxlo_hbm, rxhi_hbm, src_hbm, dst_hbm, ea_hbm,
                 outn_hbm, oute_hbm,
                 src_v, dst_v, rows0, rows1, ebuf0, ebuf1, zbuf, zbuf16,
                 accn_sh, acce_sh, gsem0, gsem1, esem0, esem1):
    cid = lax.axis_index("c")
    sid = lax.axis_index("s")

    zero16 = jnp.zeros((16,), jnp.float32)

    def _zrow(i, carry):
        for k in range(DH // 16):
            zbuf[i, pl.ds(k * 16, 16)] = zero16
        zbuf16[i, :] = zero16
        return carry

    @pl.when(sid < NZ_S)
    def _zero_fill():
        lax.fori_loop(0, ZROWS, _zrow, 0)
        for j in range(ROWS_PER_S // ZROWS):
            r0 = sid * ROWS_PER_S + j * ZROWS
            pltpu.sync_copy(zbuf, accn_sh.at[pl.ds(r0, ZROWS)])
            pltpu.sync_copy(zbuf16, acce_sh.at[pl.ds(r0, ZROWS)])

    plsc.subcore_barrier()

    pltpu.sync_copy(src_hbm.at[sid], src_v)
    pltpu.sync_copy(dst_hbm.at[sid], dst_v)

    rows_b = (rows0, rows1)
    ebuf_b = (ebuf0, ebuf1)
    gsem_b = (gsem0, gsem1)
    esem_b = (esem0, esem1)

    def _issue_gather(j, buf, sem):
        @pl.when(cid == 0)
        def _g_lo():
            pltpu.async_copy(rxlo_hbm.at[src_v.at[j]], buf, sem)

        @pl.when(cid == 1)
        def _g_hi():
            pltpu.async_copy(rxhi_hbm.at[src_v.at[j]], buf, sem)

    def _issue_edge(j, buf, sem):
        @pl.when((j // EDGE_HALF) == cid)
        def _e():
            pltpu.async_copy(ea_hbm.at[sid, j], buf, sem)

    _issue_gather(0, rows0, gsem0)
    _issue_edge(0, ebuf0, esem0)

    def _pair(s, carry):
        for b in range(2):
            j = 2 * s + b
            rows_cur, ebuf_cur = rows_b[b], ebuf_b[b]
            gsem_cur, esem_cur = gsem_b[b], esem_b[b]

            @pl.when(j + 1 < N_CHUNKS)
            def _issue_next():
                _issue_gather(j + 1, rows_b[1 - b], gsem_b[1 - b])
                _issue_edge(j + 1, ebuf_b[1 - b], esem_b[1 - b])

            pltpu.make_async_copy(rxlo_hbm.at[src_v.at[j]], rows_cur,
                                  gsem_cur).wait()
            pltpu.sync_copy(rows_cur, accn_sh.at[dst_v.at[j]], add=True)

            @pl.when((j // EDGE_HALF) == cid)
            def _edge_part():
                pltpu.make_async_copy(ea_hbm.at[sid, j], ebuf_cur,
                                      esem_cur).wait()
                for i in range(CHUNK):
                    ebuf_cur[i, :] = jnp.maximum(ebuf_cur[i, :], 0.0)
                pltpu.sync_copy(ebuf_cur, acce_sh.at[dst_v.at[j]], add=True)

        return carry

    lax.fori_loop(0, N_CHUNKS // 2, _pair, 0)
    plsc.subcore_barrier()

    @pl.when(sid < NZ_S)
    def _copy_out():
        r0 = sid * ROWS_PER_S
        pltpu.sync_copy(accn_sh.at[pl.ds(r0, ROWS_PER_S)],
                        outn_hbm.at[cid, pl.ds(r0, ROWS_PER_S)])
        pltpu.sync_copy(acce_sh.at[pl.ds(r0, ROWS_PER_S)],
                        oute_hbm.at[cid, pl.ds(r0, ROWS_PER_S)])


_sc_agg = functools.partial(
    pl.kernel,
    out_type=(jax.ShapeDtypeStruct((NC, N_NODES, DH), jnp.float32),
              jax.ShapeDtypeStruct((NC, N_NODES, D_EDGE), jnp.float32)),
    mesh=plsc.VectorSubcoreMesh(core_axis_name="c", subcore_axis_name="s"),
    compiler_params=pltpu.CompilerParams(use_tc_tiling_on_sc=False),
    scratch_types=[
        pltpu.VMEM((N_CHUNKS, CHUNK), jnp.int32),
        pltpu.VMEM((N_CHUNKS, CHUNK), jnp.int32),
        pltpu.VMEM((CHUNK, DH), jnp.float32),
        pltpu.VMEM((CHUNK, DH), jnp.float32),
        pltpu.VMEM((CHUNK, D_EDGE), jnp.float32),
        pltpu.VMEM((CHUNK, D_EDGE), jnp.float32),
        pltpu.VMEM((ZROWS, DH), jnp.float32),
        pltpu.VMEM((ZROWS, D_EDGE), jnp.float32),
        pltpu.VMEM_SHARED((N_NODES, DH), jnp.float32),
        pltpu.VMEM_SHARED((N_NODES, D_EDGE), jnp.float32),
        pltpu.SemaphoreType.DMA,
        pltpu.SemaphoreType.DMA,
        pltpu.SemaphoreType.DMA,
        pltpu.SemaphoreType.DMA,
    ],
)(_sc_agg_body)


def _relu_split_body(x_ref, olo_ref, ohi_ref):
    rx = jnp.maximum(x_ref[...], 0.0)
    olo_ref[...] = rx[:, :DH]
    ohi_ref[...] = rx[:, DH:]


def _mlp_body(an0, an1, ae0, ae1, xlo, xhi, w1lo, w1hi, w1e, b1, w2, b2,
              o_ref):
    hlo = an0[0] + xlo[...]
    hhi = an1[0] + xhi[...]
    he = ae0[0] + ae1[0]
    h1 = jnp.dot(hlo, w1lo[...], preferred_element_type=jnp.float32)
    h1 += jnp.dot(hhi, w1hi[...], preferred_element_type=jnp.float32)
    h1 += jnp.dot(he, w1e[...], preferred_element_type=jnp.float32)
    h1 = jnp.maximum(h1 + b1[...], 0.0)
    o_ref[...] = jnp.dot(h1, w2[...], preferred_element_type=jnp.float32) + b2[...]


def kernel(x, edge_index, edge_attr, W1, b1, W2, b2):
    src = edge_index[0].astype(jnp.int32).reshape(NS, N_CHUNKS, CHUNK)
    dst = edge_index[1].astype(jnp.int32).reshape(NS, N_CHUNKS, CHUNK)
    ea = edge_attr.reshape(NS, N_CHUNKS, CHUNK, D_EDGE)

    rx_lo, rx_hi = pl.pallas_call(
        _relu_split_body,
        out_shape=(jax.ShapeDtypeStruct((N_NODES, DH), jnp.float32),
                   jax.ShapeDtypeStruct((N_NODES, DH), jnp.float32)),
    )(x)

    accn, acce = _sc_agg(rx_lo, rx_hi, src, dst, ea)

    rows_blk = 1000
    grid = (N_NODES // rows_blk,)
    h_spec = pl.BlockSpec((1, rows_blk, DH), lambda i: (0, i, 0))
    h_spec1 = pl.BlockSpec((1, rows_blk, DH), lambda i: (1, i, 0))
    e_spec = pl.BlockSpec((1, rows_blk, D_EDGE), lambda i: (0, i, 0))
    e_spec1 = pl.BlockSpec((1, rows_blk, D_EDGE), lambda i: (1, i, 0))
    x_spec = pl.BlockSpec((rows_blk, DH), lambda i: (i, 0))
    out = pl.pallas_call(
        _mlp_body,
        out_shape=jax.ShapeDtypeStruct((N_NODES, OUT), jnp.float32),
        grid=grid,
        in_specs=[
            h_spec, h_spec1, e_spec, e_spec1, x_spec, x_spec,
            pl.BlockSpec((DH, HIDDEN), lambda i: (0, 0)),
            pl.BlockSpec((DH, HIDDEN), lambda i: (0, 0)),
            pl.BlockSpec((D_EDGE, HIDDEN), lambda i: (0, 0)),
            pl.BlockSpec((1, HIDDEN), lambda i: (0, 0)),
            pl.BlockSpec((HIDDEN, OUT), lambda i: (0, 0)),
            pl.BlockSpec((1, OUT), lambda i: (0, 0)),
        ],
        out_specs=pl.BlockSpec((rows_blk, OUT), lambda i: (i, 0)),
    )(accn, accn, acce, acce, x[:, :DH], x[:, DH:],
      W1[:DH], W1[DH:D_FEAT], W1[D_FEAT:], b1.reshape(1, HIDDEN),
      W2, b2.reshape(1, OUT))
    return out


# speedup vs baseline: 7.2963x; 1.0963x over previous
import functools

import jax
import jax.numpy as jnp
from jax import lax
from jax.experimental import pallas as pl
from jax.experimental.pallas import tpu as pltpu
from jax.experimental.pallas import tpu_sc as plsc

N_NODES = 10000
N_EDGES = 320000
D_FEAT = 128
D_EDGE = 16
HIDDEN = 128
OUT = 128

NC = 2
NS = 16
DH = D_FEAT // NC
CHUNK = 128
TOT_CHUNKS = N_EDGES // CHUNK
BASE_CH = TOT_CHUNKS // NS
EXTRA = TOT_CHUNKS - BASE_CH * NS
T_MAX = BASE_CH + 1
T_LOOP = 158
UNROLL = 2
EROWS = CHUNK * D_EDGE // 128
EAP_ROWS = N_EDGES * D_EDGE // 128
IDX_ROWS = 2504
EDGE_SPLIT = TOT_CHUNKS // NC
NZ_S = 10
PK_N = N_NODES * DH // 128
PK_E = N_NODES * D_EDGE // 128
ZROWS = 125


def _sc_agg_body(rxlo_hbm, rxhi_hbm, src_hbm, dst_hbm, eap_hbm,
                 outn_hbm, oute_hbm,
                 src_v, dst_v, rows, eraw, epk, zbuf, zbuf16,
                 accn_sh, acce_sh, gsem, ssem, esem, tsem):
    cid = lax.axis_index("c")
    sid = lax.axis_index("s")

    zero16 = jnp.zeros((16,), jnp.float32)

    def _zrow(i, carry):
        for k in range(DH // 16):
            zbuf[i, pl.ds(k * 16, 16)] = zero16
        zbuf16[i, :] = zero16
        return carry

    rows_z = N_NODES // NZ_S

    @pl.when(sid < NZ_S)
    def _zero_fill():
        lax.fori_loop(0, ZROWS, _zrow, 0)
        for j in range(rows_z // ZROWS):
            r0 = sid * rows_z + j * ZROWS
            pltpu.sync_copy(zbuf, accn_sh.at[pl.ds(r0, ZROWS)])
            pltpu.sync_copy(zbuf16, acce_sh.at[pl.ds(r0, ZROWS)])

    plsc.subcore_barrier()

    accn_nodes = accn_sh
    acce_nodes = acce_sh

    start = sid * BASE_CH + jnp.minimum(sid, EXTRA)
    n_t = BASE_CH + (sid < EXTRA).astype(jnp.int32)

    pltpu.sync_copy(src_hbm.at[pl.ds(start, T_MAX)], src_v)
    pltpu.sync_copy(dst_hbm.at[pl.ds(start, T_MAX)], dst_v)

    def _issue_gather(t, q):
        @pl.when(cid == 0)
        def _g_lo():
            pltpu.async_copy(rxlo_hbm.at[src_v.at[t]], rows.at[q], gsem[q])

        @pl.when(cid == 1)
        def _g_hi():
            pltpu.async_copy(rxhi_hbm.at[src_v.at[t]], rows.at[q], gsem[q])

    def _edge_active(t):
        return ((start + t) // EDGE_SPLIT) == cid

    def _issue_edge(t, q):
        @pl.when(_edge_active(t))
        def _e():
            pltpu.async_copy(eap_hbm.at[pl.ds((start + t) * EROWS, EROWS)],
                             eraw.at[q], esem[q])

    _issue_gather(0, 0)
    _issue_edge(0, 0)

    def _step(s, carry):
        for q in range(UNROLL):
            t = UNROLL * s + q
            q1 = (q + 1) % UNROLL

            @pl.when((t >= 1) & (t - 1 < n_t))
            def _drain():
                pltpu.make_async_copy(
                    rows.at[q1], accn_nodes.at[dst_v.at[t - 1]], ssem[q1]
                ).wait()

                @pl.when(_edge_active(t - 1))
                def _drain_e():
                    pltpu.make_async_copy(
                        epk.at[q1], acce_nodes.at[dst_v.at[t - 1]], tsem[q1]
                    ).wait()

            @pl.when(t + 1 < n_t)
            def _issue_next():
                _issue_gather(t + 1, q1)
                _issue_edge(t + 1, q1)

            @pl.when(t < n_t)
            def _node_part():
                pltpu.make_async_copy(rxlo_hbm.at[src_v.at[t]], rows.at[q],
                                      gsem[q]).wait()
                pltpu.async_copy(rows.at[q], accn_nodes.at[dst_v.at[t]],
                                 ssem[q], add=True)

            @pl.when((t < n_t) & _edge_active(t))
            def _edge_part():
                pltpu.make_async_copy(
                    eap_hbm.at[pl.ds((start + t) * EROWS, EROWS)],
                    eraw.at[q], esem[q]).wait()

                def _unpack_row(r, carry):
                    for k in range(8):
                        epk[q, r * 8 + k, :] = jnp.maximum(
                            eraw[q, r, pl.ds(k * 16, 16)], 0.0)
                    return carry

                lax.fori_loop(0, EROWS, _unpack_row, 0)
                pltpu.async_copy(epk.at[q], acce_nodes.at[dst_v.at[t]],
                                 tsem[q], add=True)

        return carry

    lax.fori_loop(0, T_LOOP // UNROLL, _step, 0)
    plsc.subcore_barrier()

    @pl.when(sid < NZ_S)
    def _copy_out():
        r0 = sid * rows_z
        pltpu.sync_copy(accn_sh.at[pl.ds(r0, rows_z)],
                        outn_hbm.at[cid, pl.ds(r0, rows_z)])
        pltpu.sync_copy(acce_sh.at[pl.ds(r0, rows_z)],
                        oute_hbm.at[cid, pl.ds(r0, rows_z)])


_sc_agg = functools.partial(
    pl.kernel,
    out_type=(jax.ShapeDtypeStruct((NC, N_NODES, DH), jnp.float32),
              jax.ShapeDtypeStruct((NC, N_NODES, D_EDGE), jnp.float32)),
    mesh=plsc.VectorSubcoreMesh(core_axis_name="c", subcore_axis_name="s"),
    compiler_params=pltpu.CompilerParams(use_tc_tiling_on_sc=False, internal_scratch_in_bytes=131072),
    scratch_types=[
        pltpu.VMEM((T_MAX, CHUNK), jnp.int32),
        pltpu.VMEM((T_MAX, CHUNK), jnp.int32),
        pltpu.VMEM((UNROLL, CHUNK, DH), jnp.float32),
        pltpu.VMEM((UNROLL, EROWS, 128), jnp.float32),
        pltpu.VMEM((UNROLL, CHUNK, D_EDGE), jnp.float32),
        pltpu.VMEM((ZROWS, DH), jnp.float32),
        pltpu.VMEM((ZROWS, D_EDGE), jnp.float32),
        pltpu.VMEM_SHARED((N_NODES, DH), jnp.float32),
        pltpu.VMEM_SHARED((N_NODES, D_EDGE), jnp.float32),
        [pltpu.SemaphoreType.DMA] * UNROLL,
        [pltpu.SemaphoreType.DMA] * UNROLL,
        [pltpu.SemaphoreType.DMA] * UNROLL,
        [pltpu.SemaphoreType.DMA] * UNROLL,
    ],
)(_sc_agg_body)


def _prep_body(x_ref, rxlo_ref, rxhi_ref):
    rx = jnp.maximum(x_ref[...], 0.0)
    rxlo_ref[...] = rx[:, :DH]
    rxhi_ref[...] = rx[:, DH:]


def _mlp_body(an, ae, xlo, xhi, w1lo, w1hi, w1e, b1, w2, b2, o_ref):
    hlo = an[0] + xlo[...]
    hhi = an[1] + xhi[...]
    he = ae[0] + ae[1]
    h1 = jnp.dot(hlo, w1lo[...], preferred_element_type=jnp.float32)
    h1 += jnp.dot(hhi, w1hi[...], preferred_element_type=jnp.float32)
    h1 += jnp.dot(he, w1e[...], preferred_element_type=jnp.float32)
    h1 = jnp.maximum(h1 + b1[...], 0.0)
    o_ref[...] = jnp.dot(h1, w2[...], preferred_element_type=jnp.float32) + b2[...]


def kernel(x, edge_index, edge_attr, W1, b1, W2, b2):
    src = edge_index[0].astype(jnp.int32).reshape(TOT_CHUNKS, CHUNK)
    dst = edge_index[1].astype(jnp.int32).reshape(TOT_CHUNKS, CHUNK)
    pad = ((0, IDX_ROWS - TOT_CHUNKS), (0, 0))
    src = jnp.pad(src, pad)
    dst = jnp.pad(dst, pad)

    eap = edge_attr.reshape(EAP_ROWS, 128)
    rx_lo, rx_hi = pl.pallas_call(
        _prep_body,
        out_shape=(jax.ShapeDtypeStruct((N_NODES, DH), jnp.float32),
                   jax.ShapeDtypeStruct((N_NODES, DH), jnp.float32)),
    )(x)

    accn, acce = _sc_agg(rx_lo, rx_hi, src, dst, eap)

    out = pl.pallas_call(
        _mlp_body,
        out_shape=jax.ShapeDtypeStruct((N_NODES, OUT), jnp.float32),
    )(accn, acce, x[:, :DH], x[:, DH:],
      W1[:DH], W1[DH:D_FEAT], W1[D_FEAT:], b1.reshape(1, HIDDEN),
      W2, b2.reshape(1, OUT))
    return out
